# baseline (speedup 1.0000x reference)
import jax
import jax.numpy as jnp
from jax import lax
from jax.experimental import pallas as pl
from jax.experimental.pallas import tpu as pltpu
from jax.experimental.pallas import tpu_sc as plsc

_N = 10000
_E = 320000
_D = 128
_DE = 16
_C = 2
_W272 = 2 * _D + _DE

_NC = 2
_NS = 16
_NW = _NC * _NS
_CH = 128
_NCHUNK = _E // _CH
_NFULLT = _NCHUNK // _NW
_TAILW = _NCHUNK - _NFULLT * _NW
_L = 16


def _sc_body(src_hbm, dst_hbm, emb_hbm, nproj_hbm,
             out_hbm, psc_hbm,
             ixs0, ixd0, ixs1, ixd1, rws0, rws1, rwd0, rwd1, scv0, scv1,
             nsv,
             six0, six1, sgs0, sgs1, sgd0, sgd1, sw0, sw1):
  wid = lax.axis_index("s") * _NC + lax.axis_index("c")
  pltpu.sync_copy(nproj_hbm, nsv)

  bufs = ((ixs0, ixd0, rws0, rwd0, scv0, six0, sgs0, sgd0, sw0),
          (ixs1, ixd1, rws1, rwd1, scv1, six1, sgs1, sgd1, sw1))

  def cbase(t):
    return (wid + t * _NW) * _CH

  def idx_descs(t, bs):
    ixs, ixd, six = bs[0], bs[1], bs[5]
    base = cbase(t)
    return (pltpu.make_async_copy(src_hbm.at[pl.ds(base, _CH)], ixs, six),
            pltpu.make_async_copy(dst_hbm.at[pl.ds(base, _CH)], ixd, six))

  def issue(t, bs):
    ixs, ixd, rws, rwd = bs[0], bs[1], bs[2], bs[3]
    sgs, sgd = bs[6], bs[7]
    pltpu.async_copy(emb_hbm.at[ixs], rws, sgs)
    pltpu.async_copy(emb_hbm.at[ixd], rwd, sgd)

  def scores(ixs, ixd, scv):
    for i in range(_CH // _L):
      si = ixs[pl.ds(i * _L, _L)] * 4
      di = ixd[pl.ds(i * _L, _L)] * 4
      scv[0, pl.ds(i * _L, _L)] = (
          plsc.load_gather(nsv, [si]) + plsc.load_gather(nsv, [di + 2]))
      scv[1, pl.ds(i * _L, _L)] = (
          plsc.load_gather(nsv, [si + 1]) + plsc.load_gather(nsv, [di + 3]))

  def write_descs(t, bs):
    rws, rwd, scv, sw = bs[2], bs[3], bs[4], bs[8]
    base = cbase(t)
    return (
        pltpu.make_async_copy(rws, out_hbm.at[pl.ds(base, _CH), pl.ds(0, _D)], sw),
        pltpu.make_async_copy(rwd, out_hbm.at[pl.ds(base, _CH), pl.ds(_D, _D)], sw),
        pltpu.make_async_copy(scv, psc_hbm.at[:, pl.ds(base, _CH)], sw),
    )

  def finish(t, bs):
    ixs, ixd, rws, rwd, scv = bs[0], bs[1], bs[2], bs[3], bs[4]
    sgs, sgd = bs[6], bs[7]
    scores(ixs, ixd, scv)
    d1, d2, d3 = write_descs(t, bs)
    d3.start()
    pltpu.make_async_copy(emb_hbm.at[ixs], rws, sgs).wait()
    d1.start()
    pltpu.make_async_copy(emb_hbm.at[ixd], rwd, sgd).wait()
    d2.start()

  def drain(t, bs):
    for d in write_descs(t, bs):
      d.wait()

  for d in idx_descs(0, bufs[0]):
    d.start()

  def body(s, carry):
    for b in (0, 1):
      t = s * 2 + b
      cur, prev = bufs[b], bufs[1 - b]

      @pl.when(t >= 2)
      def _():
        drain(t - 2, cur)

      @pl.when(t >= 1)
      def _():
        finish(t - 1, prev)

      for d in idx_descs(t, cur):
        d.wait()
      issue(t, cur)

      @pl.when(jnp.logical_or(t + 1 < _NFULLT, wid < _TAILW))
      def _():
        for d in idx_descs(t + 1, prev):
          d.start()
    return carry

  lax.fori_loop(0, _NFULLT // 2, body, 0)
  t_last = _NFULLT
  drain(_NFULLT - 2, bufs[0])
  finish(_NFULLT - 1, bufs[1])

  @pl.when(wid < _TAILW)
  def _():
    for d in idx_descs(t_last, bufs[0]):
      d.wait()
    issue(t_last, bufs[0])

  drain(_NFULLT - 1, bufs[1])

  @pl.when(wid < _TAILW)
  def _():
    finish(t_last, bufs[0])
    drain(t_last, bufs[0])


def _nproj_body(x_ref, w_ref, o_ref):
  o_ref[...] = jnp.dot(x_ref[...], w_ref[...],
                       preferred_element_type=jnp.float32)


def _epi_body(eft_ref, w_ref, psc_ref, osc_ref):
  proj = lax.dot_general(w_ref[...], eft_ref[...],
                         dimension_numbers=(((1,), (0,)), ((), ())),
                         preferred_element_type=jnp.float32)
  osc_ref[...] = psc_ref[...] + proj


def kernel(edge_index, edge_feat, node_emb, weight):
  w_nodes = jnp.concatenate([weight[:, :_D].T, weight[:, _D:2 * _D].T], axis=1)

  node_proj = pl.pallas_call(
      _nproj_body,
      out_shape=jax.ShapeDtypeStruct((_N, 2 * _C), jnp.float32),
  )(node_emb, w_nodes)

  sc_fn = pl.kernel(
      _sc_body,
      out_type=(jax.ShapeDtypeStruct((_E, 2 * _D), jnp.float32),
                jax.ShapeDtypeStruct((_C, _E), jnp.float32)),
      mesh=plsc.VectorSubcoreMesh(core_axis_name="c", subcore_axis_name="s",
                                  num_cores=_NC, num_subcores=_NS),
      compiler_params=pltpu.CompilerParams(needs_layout_passes=False,
                                           use_tc_tiling_on_sc=True),
      scratch_types=[
          pltpu.VMEM((_CH,), jnp.int32),
          pltpu.VMEM((_CH,), jnp.int32),
          pltpu.VMEM((_CH,), jnp.int32),
          pltpu.VMEM((_CH,), jnp.int32),
          pltpu.VMEM((_CH, _D), jnp.float32),
          pltpu.VMEM((_CH, _D), jnp.float32),
          pltpu.VMEM((_CH, _D), jnp.float32),
          pltpu.VMEM((_CH, _D), jnp.float32),
          pltpu.VMEM((_C, _CH), jnp.float32),
          pltpu.VMEM((_C, _CH), jnp.float32),
          pltpu.VMEM((_N * 2 * _C,), jnp.float32),
          pltpu.SemaphoreType.DMA,
          pltpu.SemaphoreType.DMA,
          pltpu.SemaphoreType.DMA,
          pltpu.SemaphoreType.DMA,
          pltpu.SemaphoreType.DMA,
          pltpu.SemaphoreType.DMA,
          pltpu.SemaphoreType.DMA,
          pltpu.SemaphoreType.DMA,
      ],
  )
  ee_main, psc = sc_fn(edge_index[0], edge_index[1], node_emb,
                       node_proj.reshape(-1))
  edge_embeds = jnp.concatenate([ee_main, edge_feat], axis=1)

  _EB = 12800
  osc = pl.pallas_call(
      _epi_body,
      grid=(_E // _EB,),
      in_specs=[pl.BlockSpec((_DE, _EB), lambda i: (0, i)),
                pl.BlockSpec((_C, _DE), lambda i: (0, 0)),
                pl.BlockSpec((_C, _EB), lambda i: (0, i))],
      out_specs=pl.BlockSpec((_C, _EB), lambda i: (0, i)),
      out_shape=jax.ShapeDtypeStruct((_C, _E), jnp.float32),
  )(edge_feat.T, weight[:, 2 * _D:], psc)

  return osc.T, edge_embeds


# speedup vs baseline: 1.0296x; 1.0296x over previous
import jax
import jax.numpy as jnp
from jax import lax
from jax.experimental import pallas as pl
from jax.experimental.pallas import tpu as pltpu
from jax.experimental.pallas import tpu_sc as plsc

_N = 10000
_E = 320000
_D = 128
_DE = 16
_C = 2
_W272 = 2 * _D + _DE

_NC = 2
_NS = 16
_NW = _NC * _NS
_CH = 128
_NCHUNK = _E // _CH
_NFULLT = _NCHUNK // _NW
_TAILW = _NCHUNK - _NFULLT * _NW
_L = 16


def _sc_body(src_hbm, dst_hbm, emb_hbm, ef_hbm, nproj_hbm,
             out_hbm, psc_hbm,
             ixs0, ixd0, ixs1, ixd1, rws0, rws1, rwd0, rwd1, scv0, scv1,
             efv, nsv,
             six0, six1, sgs0, sgs1, sgd0, sgd1, sw0, sw1, sef):
  wid = lax.axis_index("s") * _NC + lax.axis_index("c")
  pltpu.sync_copy(nproj_hbm, nsv)

  bufs = ((ixs0, ixd0, rws0, rwd0, scv0, six0, sgs0, sgd0, sw0),
          (ixs1, ixd1, rws1, rwd1, scv1, six1, sgs1, sgd1, sw1))

  def cbase(t):
    return (wid + t * _NW) * _CH

  def idx_descs(t, bs):
    ixs, ixd, six = bs[0], bs[1], bs[5]
    base = cbase(t)
    return (pltpu.make_async_copy(src_hbm.at[pl.ds(base, _CH)], ixs, six),
            pltpu.make_async_copy(dst_hbm.at[pl.ds(base, _CH)], ixd, six))

  def issue(t, bs):
    ixs, ixd, rws, rwd = bs[0], bs[1], bs[2], bs[3]
    sgs, sgd = bs[6], bs[7]
    pltpu.async_copy(emb_hbm.at[ixs], rws, sgs)
    pltpu.async_copy(emb_hbm.at[ixd], rwd, sgd)

  def scores(ixs, ixd, scv):
    for i in range(_CH // _L):
      si = ixs[pl.ds(i * _L, _L)] * 4
      di = ixd[pl.ds(i * _L, _L)] * 4
      scv[0, pl.ds(i * _L, _L)] = (
          plsc.load_gather(nsv, [si]) + plsc.load_gather(nsv, [di + 2]))
      scv[1, pl.ds(i * _L, _L)] = (
          plsc.load_gather(nsv, [si + 1]) + plsc.load_gather(nsv, [di + 3]))

  def write_descs(t, bs):
    rws, rwd, scv, sw = bs[2], bs[3], bs[4], bs[8]
    base = cbase(t)
    return (
        pltpu.make_async_copy(rws, out_hbm.at[pl.ds(base, _CH), pl.ds(0, _D)], sw),
        pltpu.make_async_copy(rwd, out_hbm.at[pl.ds(base, _CH), pl.ds(_D, _D)], sw),
        pltpu.make_async_copy(scv, psc_hbm.at[:, pl.ds(base, _CH)], sw),
        pltpu.make_async_copy(efv, out_hbm.at[pl.ds(base, _CH),
                                              pl.ds(2 * _D, _DE)], sw),
    )

  def finish(t, bs):
    ixs, ixd, rws, rwd, scv = bs[0], bs[1], bs[2], bs[3], bs[4]
    sgs, sgd = bs[6], bs[7]
    base = cbase(t)
    efl = pltpu.async_copy(ef_hbm.at[pl.ds(base, _CH)], efv, sef)
    scores(ixs, ixd, scv)
    d1, d2, d3, d4 = write_descs(t, bs)
    d3.start()
    efl.wait()
    d4.start()
    pltpu.make_async_copy(emb_hbm.at[ixs], rws, sgs).wait()
    d1.start()
    pltpu.make_async_copy(emb_hbm.at[ixd], rwd, sgd).wait()
    d2.start()

  def drain(t, bs):
    for d in write_descs(t, bs):
      d.wait()

  for d in idx_descs(0, bufs[0]):
    d.start()

  def body(s, carry):
    for b in (0, 1):
      t = s * 2 + b
      cur, prev = bufs[b], bufs[1 - b]

      @pl.when(t >= 2)
      def _():
        drain(t - 2, cur)

      @pl.when(t >= 1)
      def _():
        finish(t - 1, prev)

      for d in idx_descs(t, cur):
        d.wait()
      issue(t, cur)

      @pl.when(jnp.logical_or(t + 1 < _NFULLT, wid < _TAILW))
      def _():
        for d in idx_descs(t + 1, prev):
          d.start()
    return carry

  lax.fori_loop(0, _NFULLT // 2, body, 0)
  t_last = _NFULLT
  drain(_NFULLT - 2, bufs[0])
  finish(_NFULLT - 1, bufs[1])

  @pl.when(wid < _TAILW)
  def _():
    for d in idx_descs(t_last, bufs[0]):
      d.wait()
    issue(t_last, bufs[0])

  drain(_NFULLT - 1, bufs[1])

  @pl.when(wid < _TAILW)
  def _():
    finish(t_last, bufs[0])
    drain(t_last, bufs[0])


def _nproj_body(x_ref, w_ref, o_ref):
  o_ref[...] = jnp.dot(x_ref[...], w_ref[...],
                       preferred_element_type=jnp.float32)


def _epi_body(eft_ref, w_ref, psc_ref, osc_ref):
  proj = lax.dot_general(w_ref[...], eft_ref[...],
                         dimension_numbers=(((1,), (0,)), ((), ())),
                         preferred_element_type=jnp.float32)
  osc_ref[...] = psc_ref[...] + proj


def kernel(edge_index, edge_feat, node_emb, weight):
  w_nodes = jnp.concatenate([weight[:, :_D].T, weight[:, _D:2 * _D].T], axis=1)

  node_proj = pl.pallas_call(
      _nproj_body,
      out_shape=jax.ShapeDtypeStruct((_N, 2 * _C), jnp.float32),
  )(node_emb, w_nodes)

  sc_fn = pl.kernel(
      _sc_body,
      out_type=(jax.ShapeDtypeStruct((_E, _W272), jnp.float32),
                jax.ShapeDtypeStruct((_C, _E), jnp.float32)),
      mesh=plsc.VectorSubcoreMesh(core_axis_name="c", subcore_axis_name="s",
                                  num_cores=_NC, num_subcores=_NS),
      compiler_params=pltpu.CompilerParams(needs_layout_passes=False,
                                           use_tc_tiling_on_sc=True),
      scratch_types=[
          pltpu.VMEM((_CH,), jnp.int32),
          pltpu.VMEM((_CH,), jnp.int32),
          pltpu.VMEM((_CH,), jnp.int32),
          pltpu.VMEM((_CH,), jnp.int32),
          pltpu.VMEM((_CH, _D), jnp.float32),
          pltpu.VMEM((_CH, _D), jnp.float32),
          pltpu.VMEM((_CH, _D), jnp.float32),
          pltpu.VMEM((_CH, _D), jnp.float32),
          pltpu.VMEM((_C, _CH), jnp.float32),
          pltpu.VMEM((_C, _CH), jnp.float32),
          pltpu.VMEM((_CH, _DE), jnp.float32),
          pltpu.VMEM((_N * 2 * _C,), jnp.float32),
          pltpu.SemaphoreType.DMA,
          pltpu.SemaphoreType.DMA,
          pltpu.SemaphoreType.DMA,
          pltpu.SemaphoreType.DMA,
          pltpu.SemaphoreType.DMA,
          pltpu.SemaphoreType.DMA,
          pltpu.SemaphoreType.DMA,
          pltpu.SemaphoreType.DMA,
          pltpu.SemaphoreType.DMA,
      ],
  )
  edge_embeds, psc = sc_fn(edge_index[0], edge_index[1], node_emb,
                           edge_feat, node_proj.reshape(-1))

  _EB = 12800
  osc = pl.pallas_call(
      _epi_body,
      grid=(_E // _EB,),
      in_specs=[pl.BlockSpec((_DE, _EB), lambda i: (0, i)),
                pl.BlockSpec((_C, _DE), lambda i: (0, 0)),
                pl.BlockSpec((_C, _EB), lambda i: (0, i))],
      out_specs=pl.BlockSpec((_C, _EB), lambda i: (0, i)),
      out_shape=jax.ShapeDtypeStruct((_C, _E), jnp.float32),
  )(edge_feat.T, weight[:, 2 * _D:], psc)

  return osc.T, edge_embeds


# speedup vs baseline: 1.0567x; 1.0264x over previous
import jax
import jax.numpy as jnp
from jax import lax
from jax.experimental import pallas as pl
from jax.experimental.pallas import tpu as pltpu
from jax.experimental.pallas import tpu_sc as plsc

_N = 10000
_E = 320000
_D = 128
_DE = 16
_C = 2
_W272 = 2 * _D + _DE

_NC = 2
_NS = 16
_NW = _NC * _NS
_CH = 128
_NCHUNK = _E // _CH
_NFULLT = _NCHUNK // _NW
_TAILW = _NCHUNK - _NFULLT * _NW
_L = 16


def _sc_body(src_hbm, dst_hbm, emb_hbm, ef_hbm, nproj_hbm,
             out_hbm, psc_hbm,
             ixs0, ixd0, ixs1, ixd1, rws0, rws1, rwd0, rwd1, scv0, scv1,
             efv, nsv,
             six0, six1, sgs0, sgs1, sgd0, sgd1, sw0, sw1, sef, sefw):
  wid = lax.axis_index("s") * _NC + lax.axis_index("c")
  pltpu.sync_copy(nproj_hbm, nsv)

  bufs = ((ixs0, ixd0, rws0, rwd0, scv0, six0, sgs0, sgd0, sw0),
          (ixs1, ixd1, rws1, rwd1, scv1, six1, sgs1, sgd1, sw1))

  def cbase(t):
    return (wid + t * _NW) * _CH

  def idx_descs(t, bs):
    ixs, ixd, six = bs[0], bs[1], bs[5]
    base = cbase(t)
    return (pltpu.make_async_copy(src_hbm.at[pl.ds(base, _CH)], ixs, six),
            pltpu.make_async_copy(dst_hbm.at[pl.ds(base, _CH)], ixd, six))

  def issue(t, bs):
    ixs, ixd, rws, rwd = bs[0], bs[1], bs[2], bs[3]
    sgs, sgd = bs[6], bs[7]
    pltpu.async_copy(emb_hbm.at[ixs], rws, sgs)
    pltpu.async_copy(emb_hbm.at[ixd], rwd, sgd)

  def scores(ixs, ixd, scv):
    for i in range(_CH // _L):
      si = ixs[pl.ds(i * _L, _L)] * 4
      di = ixd[pl.ds(i * _L, _L)] * 4
      scv[0, pl.ds(i * _L, _L)] = (
          plsc.load_gather(nsv, [si]) + plsc.load_gather(nsv, [di + 2]))
      scv[1, pl.ds(i * _L, _L)] = (
          plsc.load_gather(nsv, [si + 1]) + plsc.load_gather(nsv, [di + 3]))

  def write_descs(t, bs):
    rws, rwd, scv, sw = bs[2], bs[3], bs[4], bs[8]
    base = cbase(t)
    return (
        pltpu.make_async_copy(rws, out_hbm.at[pl.ds(base, _CH), pl.ds(0, _D)], sw),
        pltpu.make_async_copy(rwd, out_hbm.at[pl.ds(base, _CH), pl.ds(_D, _D)], sw),
        pltpu.make_async_copy(scv, psc_hbm.at[:, pl.ds(base, _CH)], sw),
    )

  def ef_write_desc(t):
    return pltpu.make_async_copy(
        efv, out_hbm.at[pl.ds(cbase(t), _CH), pl.ds(2 * _D, _DE)], sefw)

  def ef_load(t):
    pltpu.async_copy(ef_hbm.at[pl.ds(cbase(t), _CH)], efv, sef)

  def ef_load_wait(t):
    pltpu.make_async_copy(ef_hbm.at[pl.ds(cbase(t), _CH)], efv, sef).wait()

  def finish(t, bs):
    ixs, ixd, rws, rwd, scv = bs[0], bs[1], bs[2], bs[3], bs[4]
    sgs, sgd = bs[6], bs[7]
    scores(ixs, ixd, scv)
    d1, d2, d3 = write_descs(t, bs)
    d3.start()
    ef_load_wait(t)
    ef_write_desc(t).start()
    pltpu.make_async_copy(emb_hbm.at[ixs], rws, sgs).wait()
    d1.start()
    pltpu.make_async_copy(emb_hbm.at[ixd], rwd, sgd).wait()
    d2.start()

  def drain(t, bs):
    for d in write_descs(t, bs):
      d.wait()

  for d in idx_descs(0, bufs[0]):
    d.start()
  ef_load(0)

  def body(s, carry):
    for b in (0, 1):
      t = s * 2 + b
      cur, prev = bufs[b], bufs[1 - b]

      @pl.when(t >= 2)
      def _():
        drain(t - 2, cur)

      @pl.when(t >= 1)
      def _():
        finish(t - 1, prev)

      for d in idx_descs(t, cur):
        d.wait()
      issue(t, cur)

      @pl.when(jnp.logical_or(t + 1 < _NFULLT, wid < _TAILW))
      def _():
        for d in idx_descs(t + 1, prev):
          d.start()

      @pl.when(t >= 1)
      def _():
        ef_write_desc(t - 1).wait()

      @pl.when(jnp.logical_or(t + 1 < _NFULLT, wid < _TAILW))
      def _():
        ef_load(t + 1)
    return carry

  lax.fori_loop(0, _NFULLT // 2, body, 0)
  t_last = _NFULLT
  drain(_NFULLT - 2, bufs[0])
  finish(_NFULLT - 1, bufs[1])

  @pl.when(wid < _TAILW)
  def _():
    for d in idx_descs(t_last, bufs[0]):
      d.wait()
    issue(t_last, bufs[0])

  ef_write_desc(_NFULLT - 1).wait()

  @pl.when(wid < _TAILW)
  def _():
    ef_load(t_last)

  drain(_NFULLT - 1, bufs[1])

  @pl.when(wid < _TAILW)
  def _():
    finish(t_last, bufs[0])
    drain(t_last, bufs[0])
    ef_write_desc(t_last).wait()


def _nproj_body(x_ref, w_ref, o_ref):
  o_ref[...] = jnp.dot(x_ref[...], w_ref[...],
                       preferred_element_type=jnp.float32)


def _epi_body(eft_ref, w_ref, psc_ref, osc_ref):
  proj = lax.dot_general(w_ref[...], eft_ref[...],
                         dimension_numbers=(((1,), (0,)), ((), ())),
                         preferred_element_type=jnp.float32)
  osc_ref[...] = psc_ref[...] + proj


def kernel(edge_index, edge_feat, node_emb, weight):
  w_nodes = jnp.concatenate([weight[:, :_D].T, weight[:, _D:2 * _D].T], axis=1)

  node_proj = pl.pallas_call(
      _nproj_body,
      out_shape=jax.ShapeDtypeStruct((_N, 2 * _C), jnp.float32),
  )(node_emb, w_nodes)

  sc_fn = pl.kernel(
      _sc_body,
      out_type=(jax.ShapeDtypeStruct((_E, _W272), jnp.float32),
                jax.ShapeDtypeStruct((_C, _E), jnp.float32)),
      mesh=plsc.VectorSubcoreMesh(core_axis_name="c", subcore_axis_name="s",
                                  num_cores=_NC, num_subcores=_NS),
      compiler_params=pltpu.CompilerParams(needs_layout_passes=False,
                                           use_tc_tiling_on_sc=True),
      scratch_types=[
          pltpu.VMEM((_CH,), jnp.int32),
          pltpu.VMEM((_CH,), jnp.int32),
          pltpu.VMEM((_CH,), jnp.int32),
          pltpu.VMEM((_CH,), jnp.int32),
          pltpu.VMEM((_CH, _D), jnp.float32),
          pltpu.VMEM((_CH, _D), jnp.float32),
          pltpu.VMEM((_CH, _D), jnp.float32),
          pltpu.VMEM((_CH, _D), jnp.float32),
          pltpu.VMEM((_C, _CH), jnp.float32),
          pltpu.VMEM((_C, _CH), jnp.float32),
          pltpu.VMEM((_CH, _DE), jnp.float32),
          pltpu.VMEM((_N * 2 * _C,), jnp.float32),
          pltpu.SemaphoreType.DMA,
          pltpu.SemaphoreType.DMA,
          pltpu.SemaphoreType.DMA,
          pltpu.SemaphoreType.DMA,
          pltpu.SemaphoreType.DMA,
          pltpu.SemaphoreType.DMA,
          pltpu.SemaphoreType.DMA,
          pltpu.SemaphoreType.DMA,
          pltpu.SemaphoreType.DMA,
          pltpu.SemaphoreType.DMA,
      ],
  )
  edge_embeds, psc = sc_fn(edge_index[0], edge_index[1], node_emb,
                           edge_feat, node_proj.reshape(-1))

  _EB = 12800
  osc = pl.pallas_call(
      _epi_body,
      grid=(_E // _EB,),
      in_specs=[pl.BlockSpec((_DE, _EB), lambda i: (0, i)),
                pl.BlockSpec((_C, _DE), lambda i: (0, 0)),
                pl.BlockSpec((_C, _EB), lambda i: (0, i))],
      out_specs=pl.BlockSpec((_C, _EB), lambda i: (0, i)),
      out_shape=jax.ShapeDtypeStruct((_C, _E), jnp.float32),
  )(edge_feat.T, weight[:, 2 * _D:], psc)

  return osc.T, edge_embeds
